# speedup vs baseline: 1.1175x; 1.1175x over previous
import jax
import jax.numpy as jnp
from jax.experimental import pallas as pl
from jax.experimental.pallas import tpu as pltpu

TOPK = 8
NUM_EXPERTS = 64
BM = 1024


def _gating_kernel(x_ref, w_ref, b_ref, w_out_ref, i_out_ref):
    x = x_ref[...]
    w = w_ref[...]
    logits = jax.lax.dot_general(
        x, w, (((1,), (1,)), ((), ())),
        preferred_element_type=jnp.float32,
    ) + b_ref[...]

    iota = jax.lax.broadcasted_iota(jnp.int32, logits.shape, 1)
    cur = logits
    vals = []
    idxs = []
    for _ in range(TOPK):
        m = jnp.max(cur, axis=1, keepdims=True)
        idx = jnp.min(
            jnp.where(cur == m, iota, NUM_EXPERTS), axis=1, keepdims=True
        )
        vals.append(m)
        idxs.append(idx)
        cur = jnp.where(iota == idx, -jnp.inf, cur)
    v = jnp.concatenate(vals, axis=1)
    ix = jnp.concatenate(idxs, axis=1)

    e = jnp.exp(v - v[:, 0:1])
    w_out_ref[...] = e / jnp.sum(e, axis=1, keepdims=True)
    i_out_ref[...] = ix


def kernel(x, W, b):
    n, d = x.shape
    b2 = b.reshape(1, NUM_EXPERTS)
    grid = (n // BM,)
    weights, indices = pl.pallas_call(
        _gating_kernel,
        grid=grid,
        in_specs=[
            pl.BlockSpec((BM, d), lambda i: (i, 0)),
            pl.BlockSpec((NUM_EXPERTS, d), lambda i: (0, 0)),
            pl.BlockSpec((1, NUM_EXPERTS), lambda i: (0, 0)),
        ],
        out_specs=[
            pl.BlockSpec((BM, TOPK), lambda i: (i, 0)),
            pl.BlockSpec((BM, TOPK), lambda i: (i, 0)),
        ],
        out_shape=[
            jax.ShapeDtypeStruct((n, TOPK), jnp.float32),
            jax.ShapeDtypeStruct((n, TOPK), jnp.int32),
        ],
    )(x, W, b2)
    return (weights, indices)


# speedup vs baseline: 1.2749x; 1.1408x over previous
import jax
import jax.numpy as jnp
from jax.experimental import pallas as pl
from jax.experimental.pallas import tpu as pltpu

TOPK = 8
NUM_EXPERTS = 64
BM = 1024

def _gating_kernel(x_ref, w_ref, b_ref, w_out_ref, i_out_ref):
    x = x_ref[...]
    w = w_ref[...]
    logits = jax.lax.dot_general(
        x, w, (((1,), (1,)), ((), ())),
        preferred_element_type=jnp.float32,
    ) + b_ref[...]

    bits = jax.lax.bitcast_convert_type(logits, jnp.int32)
    key = bits ^ (jax.lax.shift_right_arithmetic(bits, 31) & jnp.int32(0x7FFFFFFF))
    iota = jax.lax.broadcasted_iota(jnp.int32, key.shape, 1)
    key = (key & jnp.int32(~0x3F)) | (jnp.int32(NUM_EXPERTS - 1) - iota)

    cur = key
    picked = []
    for _ in range(TOPK):
        m = jnp.max(cur, axis=1, keepdims=True)
        picked.append(m)
        cur = jnp.where(cur == m, jnp.int32(-2147483648), cur)
    k = jnp.concatenate(picked, axis=1)

    ix = jnp.int32(NUM_EXPERTS - 1) - (k & jnp.int32(0x3F))
    vb = k ^ (jax.lax.shift_right_arithmetic(k, 31) & jnp.int32(0x7FFFFFFF))
    v = jax.lax.bitcast_convert_type(vb, jnp.float32)

    e = jnp.exp(v - v[:, 0:1])
    w_out_ref[...] = e / jnp.sum(e, axis=1, keepdims=True)
    i_out_ref[...] = ix


def kernel(x, W, b):
    n, d = x.shape
    b2 = b.reshape(1, NUM_EXPERTS)
    grid = (n // BM,)
    weights, indices = pl.pallas_call(
        _gating_kernel,
        grid=grid,
        in_specs=[
            pl.BlockSpec((BM, d), lambda i: (i, 0)),
            pl.BlockSpec((NUM_EXPERTS, d), lambda i: (0, 0)),
            pl.BlockSpec((1, NUM_EXPERTS), lambda i: (0, 0)),
        ],
        out_specs=[
            pl.BlockSpec((BM, TOPK), lambda i: (i, 0)),
            pl.BlockSpec((BM, TOPK), lambda i: (i, 0)),
        ],
        out_shape=[
            jax.ShapeDtypeStruct((n, TOPK), jnp.float32),
            jax.ShapeDtypeStruct((n, TOPK), jnp.int32),
        ],
    )(x, W, b2)
    return (weights, indices)


# speedup vs baseline: 1.2817x; 1.0054x over previous
import jax
import jax.numpy as jnp
from jax.experimental import pallas as pl
from jax.experimental.pallas import tpu as pltpu

TOPK = 8
NUM_EXPERTS = 64
BM = 1024

def _gating_kernel(x_ref, w_ref, b_ref, w_out_ref, i_out_ref):
    x = x_ref[...]
    w = w_ref[...]
    logits = jax.lax.dot_general(
        x, w, (((1,), (1,)), ((), ())),
        preferred_element_type=jnp.float32,
    ) + b_ref[...]

    iota_f = jax.lax.broadcasted_iota(jnp.int32, logits.shape, 1).astype(
        jnp.float32)
    cur = logits
    vals = []
    idxs = []
    for _ in range(TOPK):
        m = jnp.max(cur, axis=1, keepdims=True)
        idxf = jnp.min(
            jnp.where(cur == m, iota_f, float(NUM_EXPERTS)),
            axis=1, keepdims=True,
        )
        vals.append(m)
        idxs.append(idxf)
        cur = jnp.where(iota_f == idxf, -jnp.inf, cur)
    v = jnp.concatenate(vals, axis=1)
    ix = jnp.concatenate(idxs, axis=1).astype(jnp.int32)

    e = jnp.exp(v - v[:, 0:1])
    w_out_ref[...] = e / jnp.sum(e, axis=1, keepdims=True)
    i_out_ref[...] = ix


def kernel(x, W, b):
    n, d = x.shape
    b2 = b.reshape(1, NUM_EXPERTS)
    grid = (n // BM,)
    weights, indices = pl.pallas_call(
        _gating_kernel,
        grid=grid,
        in_specs=[
            pl.BlockSpec((BM, d), lambda i: (i, 0)),
            pl.BlockSpec((NUM_EXPERTS, d), lambda i: (0, 0)),
            pl.BlockSpec((1, NUM_EXPERTS), lambda i: (0, 0)),
        ],
        out_specs=[
            pl.BlockSpec((BM, TOPK), lambda i: (i, 0)),
            pl.BlockSpec((BM, TOPK), lambda i: (i, 0)),
        ],
        out_shape=[
            jax.ShapeDtypeStruct((n, TOPK), jnp.float32),
            jax.ShapeDtypeStruct((n, TOPK), jnp.int32),
        ],
    )(x, W, b2)
    return (weights, indices)
